# Initial kernel scaffold; baseline (speedup 1.0000x reference)
import jax
import jax.numpy as jnp
from jax.experimental import pallas as pl


def kernel(x, edge_index, edge_attr, params):
    raise NotImplementedError("write your pallas kernel here")


# speedup vs baseline: 1.7662x; 1.7662x over previous
import functools

import jax
import jax.numpy as jnp
from jax import lax
from jax.experimental import pallas as pl
from jax.experimental.pallas import tpu as pltpu
from jax.experimental.pallas import tpu_sc as plsc

N_NODES = 10000
N_EDGES = 320000
D = 128

NN_PAD = 10240
NE_PAD = 327680
NW = 32
NCH = 80
CH = 128
EPW = NCH * CH

BN = 1024
BE = 2048

_F32 = jnp.float32


def _mish(v):
    return v * jnp.tanh(jax.nn.softplus(v))


def _relu(v):
    return jnp.maximum(v, 0.0)


def _dot(a, b):
    return jnp.dot(a, b, preferred_element_type=_F32)


def _wspec(r, c):
    return pl.BlockSpec((r, c), lambda i: (0, 0))


def _rspec(rows, cols):
    return pl.BlockSpec((rows, cols), lambda i: (i, 0))


def _node_enc_body(x_ref, w0, b0, w1, b1, wa, wb, h_ref, a_ref, b_ref):
    t = _mish(_dot(x_ref[...], w0[...]) + b0[...])
    h = _dot(t, w1[...]) + b1[...]
    h_ref[...] = h
    a_ref[...] = _dot(h, wa[...])
    b_ref[...] = _dot(h, wb[...])


def _node_encode(x_p, w0, b0, w1, b1, wa, wb):
    return pl.pallas_call(
        _node_enc_body,
        grid=(NN_PAD // BN,),
        in_specs=[_rspec(BN, D), _wspec(D, D), _wspec(1, D), _wspec(D, D),
                  _wspec(1, D), _wspec(D, D), _wspec(D, D)],
        out_specs=[_rspec(BN, D)] * 3,
        out_shape=[jax.ShapeDtypeStruct((NN_PAD, D), _F32)] * 3,
    )(x_p, w0, b0, w1, b1, wa, wb)


def _edge_enc_body(ea_ref, e0, e0b, e1, e1b, wc, cb, c_ref):
    t = _mish(_dot(ea_ref[...], e0[...]) + e0b[...])
    e = _dot(t, e1[...]) + e1b[...]
    c_ref[...] = _dot(e, wc[...]) + cb[...]


def _edge_encode(ea_p, e0, e0b, e1, e1b, wc, cb):
    return pl.pallas_call(
        _edge_enc_body,
        grid=(NE_PAD // BE,),
        in_specs=[_rspec(BE, 4), _wspec(4, D), _wspec(1, D), _wspec(D, D),
                  _wspec(1, D), _wspec(D, D), _wspec(1, D)],
        out_specs=_rspec(BE, D),
        out_shape=jax.ShapeDtypeStruct((NE_PAD, D), _F32),
    )(ea_p, e0, e0b, e1, e1b, wc, cb)


def _edge_mega_body(final, ga, gb, c, w2, b2, w3, b3,
                    f0, f0b, f1, f1b, f2, f2b, t0, t0b, t1, t1b,
                    msg_ref, tail_ref):
    z1 = _relu(ga[...] + gb[...] + c[...])
    h2 = _relu(_dot(z1, w2[...]) + b2[...])
    msg = _dot(h2, w3[...]) + b3[...]
    msg_ref[...] = msg
    f = _relu(_dot(msg, f0[...]) + f0b[...])
    f = _relu(_dot(f, f1[...]) + f1b[...])
    e2 = _dot(f, f2[...]) + f2b[...]
    if final:
        d = _mish(_dot(e2, t0[...]) + t0b[...])
        tail_ref[...] = _dot(d, t1[...]) + t1b[...]
    else:
        tail_ref[...] = _dot(e2, t0[...]) + t0b[...]


def _edge_mega(final, ga, gb, c, w2, b2, w3, b3,
               f0, f0b, f1, f1b, f2, f2b, t0, t0b, t1, t1b):
    tail_w = 4 if final else D
    return pl.pallas_call(
        functools.partial(_edge_mega_body, final),
        grid=(NE_PAD // BE,),
        in_specs=[_rspec(BE, D)] * 3 + [
            _wspec(D, D), _wspec(1, D), _wspec(D, D), _wspec(1, D),
            _wspec(D, D), _wspec(1, D), _wspec(D, D), _wspec(1, D),
            _wspec(D, D), _wspec(1, D),
            _wspec(D, t0.shape[1]), _wspec(1, t0b.shape[1]),
            _wspec(t1.shape[0], t1.shape[1]), _wspec(1, t1b.shape[1])],
        out_specs=[_rspec(BE, D), _rspec(BE, tail_w)],
        out_shape=[jax.ShapeDtypeStruct((NE_PAD, D), _F32),
                   jax.ShapeDtypeStruct((NE_PAD, tail_w), _F32)],
    )(ga, gb, c, w2, b2, w3, b3, f0, f0b, f1, f1b, f2, f2b, t0, t0b, t1, t1b)


def _node_fn_body(final, h_ref, ag0, ag1, na, nb, n0b, n1, n1b, n2, n2b,
                  t0, t0b, t1, t1b, *out_refs):
    agg = ag0[...] + ag1[...]
    n = _relu(_dot(h_ref[...], na[...]) + _dot(agg, nb[...]) + n0b[...])
    n = _relu(_dot(n, n1[...]) + n1b[...])
    h2 = _dot(n, n2[...]) + n2b[...]
    if final:
        d = _mish(_dot(h2, t0[...]) + t0b[...])
        out_refs[0][...] = _dot(d, t1[...]) + t1b[...]
    else:
        out_refs[0][...] = h2
        out_refs[1][...] = _dot(h2, t0[...])
        out_refs[2][...] = _dot(h2, t1[...])


def _node_fn(final, h, ag0, ag1, na, nb, n0b, n1, n1b, n2, n2b,
             t0, t0b, t1, t1b):
    if final:
        out_specs = [_rspec(BN, 3)]
        out_shape = [jax.ShapeDtypeStruct((NN_PAD, 3), _F32)]
    else:
        out_specs = [_rspec(BN, D)] * 3
        out_shape = [jax.ShapeDtypeStruct((NN_PAD, D), _F32)] * 3
    res = pl.pallas_call(
        functools.partial(_node_fn_body, final),
        grid=(NN_PAD // BN,),
        in_specs=[_rspec(BN, D)] * 3 + [
            _wspec(D, D), _wspec(D, D), _wspec(1, D), _wspec(D, D),
            _wspec(1, D), _wspec(D, D), _wspec(1, D),
            _wspec(D, t0.shape[1]), _wspec(1, t0b.shape[1]),
            _wspec(t1.shape[0], t1.shape[1]), _wspec(1, t1b.shape[1])],
        out_specs=out_specs,
        out_shape=out_shape,
    )(h, ag0, ag1, na, nb, n0b, n1, n1b, n2, n2b, t0, t0b, t1, t1b)
    return res


def _sc_mesh():
    return plsc.VectorSubcoreMesh(core_axis_name="c", subcore_axis_name="s")


def _sc_gather(a_pad, b_pad, dst_r, src_r):
    @functools.partial(
        pl.kernel,
        out_type=[jax.ShapeDtypeStruct((NE_PAD, D), _F32),
                  jax.ShapeDtypeStruct((NE_PAD, D), _F32)],
        mesh=_sc_mesh(),
        scratch_types=[
            pltpu.VMEM((NCH, CH), jnp.int32),
            pltpu.VMEM((NCH, CH), jnp.int32),
            pltpu.VMEM((CH, D), _F32),
            pltpu.VMEM((CH, D), _F32),
            pltpu.SemaphoreType.DMA,
            pltpu.SemaphoreType.DMA,
        ],
    )
    def k(a_hbm, b_hbm, dst_hbm, src_hbm, ga_hbm, gb_hbm,
          dst_v, src_v, rows_a, rows_b, sem_a, sem_b):
        wid = lax.axis_index("s") * 2 + lax.axis_index("c")
        base = wid * EPW
        pltpu.sync_copy(dst_hbm.at[wid], dst_v)
        pltpu.sync_copy(src_hbm.at[wid], src_v)

        def chunk(j, carry):
            ca = pltpu.async_copy(a_hbm.at[dst_v.at[j]], rows_a, sem_a)
            cb = pltpu.async_copy(b_hbm.at[src_v.at[j]], rows_b, sem_b)
            ca.wait()
            cb.wait()
            pltpu.sync_copy(rows_a, ga_hbm.at[pl.ds(base + j * CH, CH)])
            pltpu.sync_copy(rows_b, gb_hbm.at[pl.ds(base + j * CH, CH)])
            return carry

        lax.fori_loop(0, NCH, chunk, 0)

    return k(a_pad, b_pad, dst_r, src_r)


def _sc_scatter(msg, idx_r, zeros_nodes):
    rows_per_tile = NN_PAD // 16

    @functools.partial(
        pl.kernel,
        out_type=jax.ShapeDtypeStruct((2, NN_PAD, D), _F32),
        mesh=_sc_mesh(),
        scratch_types=[
            pltpu.VMEM((NCH, CH), jnp.int32),
            pltpu.VMEM((CH, D), _F32),
            pltpu.VMEM_SHARED((NN_PAD, D), _F32),
        ],
    )
    def k(msg_hbm, idx_hbm, zer_hbm, out_hbm, idx_v, rows_v, acc):
        cid = lax.axis_index("c")
        sid = lax.axis_index("s")
        wid = sid * 2 + cid
        base = wid * EPW
        pltpu.sync_copy(idx_hbm.at[wid], idx_v)
        pltpu.sync_copy(zer_hbm.at[pl.ds(sid * rows_per_tile, rows_per_tile)],
                        acc.at[pl.ds(sid * rows_per_tile, rows_per_tile)])
        plsc.subcore_barrier()

        def chunk(j, carry):
            pltpu.sync_copy(msg_hbm.at[pl.ds(base + j * CH, CH)], rows_v)
            pltpu.sync_copy(rows_v, acc.at[idx_v.at[j]], add=True)
            return carry

        lax.fori_loop(0, NCH, chunk, 0)
        plsc.subcore_barrier()
        pltpu.sync_copy(acc.at[pl.ds(sid * rows_per_tile, rows_per_tile)],
                        out_hbm.at[cid, pl.ds(sid * rows_per_tile, rows_per_tile)])

    return k(msg, idx_r, zeros_nodes)


def kernel(x, edge_index, edge_attr, params):
    ne0, ne1 = params['node_encoder']
    ee0, ee1 = params['edge_encoder']
    pr1, pr2 = params['processors']
    nd0, nd1 = params['node_decoder']
    ed0, ed1 = params['edge_decoder']

    def rb(b):
        return b.reshape(1, -1)

    em1_w0, em1_b0 = pr1['edge_mlp'][0]
    em2_w0, em2_b0 = pr2['edge_mlp'][0]
    w1a_1, w1b_1, w1c_1 = em1_w0[:D], em1_w0[D:2 * D], em1_w0[2 * D:]
    w1a_2, w1b_2, w1c_2 = em2_w0[:D], em2_w0[D:2 * D], em2_w0[2 * D:]

    nf1_w0, nf1_b0 = pr1['node_fn'][0]
    nf2_w0, nf2_b0 = pr2['node_fn'][0]

    src = edge_index[0]
    dst = edge_index[1]
    pad_e = NE_PAD - N_EDGES
    src_r = jnp.pad(src, (0, pad_e), constant_values=N_NODES).reshape(NW, NCH, CH)
    dst_r = jnp.pad(dst, (0, pad_e), constant_values=N_NODES).reshape(NW, NCH, CH)
    x_p = jnp.pad(x, ((0, NN_PAD - N_NODES), (0, 0)))
    ea_p = jnp.pad(edge_attr, ((0, pad_e), (0, 0)))
    zeros_nodes = jnp.zeros((NN_PAD, D), _F32)

    h, a1, b1 = _node_encode(x_p, ne0[0], rb(ne0[1]), ne1[0], rb(ne1[1]),
                             w1a_1, w1b_1)
    c1 = _edge_encode(ea_p, ee0[0], rb(ee0[1]), ee1[0], rb(ee1[1]),
                      w1c_1, rb(em1_b0))

    ga1, gb1 = _sc_gather(a1, b1, dst_r, src_r)
    msg1, c2 = _edge_mega(
        False, ga1, gb1, c1,
        pr1['edge_mlp'][1][0], rb(pr1['edge_mlp'][1][1]),
        pr1['edge_mlp'][2][0], rb(pr1['edge_mlp'][2][1]),
        pr1['edge_fn'][0][0], rb(pr1['edge_fn'][0][1]),
        pr1['edge_fn'][1][0], rb(pr1['edge_fn'][1][1]),
        pr1['edge_fn'][2][0], rb(pr1['edge_fn'][2][1]),
        w1c_2, rb(em2_b0), w1c_2, rb(em2_b0))
    agg1 = _sc_scatter(msg1, src_r, zeros_nodes)
    h2, a2, b2 = _node_fn(
        False, h, agg1[0], agg1[1],
        nf1_w0[:D], nf1_w0[D:], rb(nf1_b0),
        pr1['node_fn'][1][0], rb(pr1['node_fn'][1][1]),
        pr1['node_fn'][2][0], rb(pr1['node_fn'][2][1]),
        w1a_2, rb(em2_b0), w1b_2, rb(em2_b0))

    ga2, gb2 = _sc_gather(a2, b2, dst_r, src_r)
    msg2, edge_out = _edge_mega(
        True, ga2, gb2, c2,
        pr2['edge_mlp'][1][0], rb(pr2['edge_mlp'][1][1]),
        pr2['edge_mlp'][2][0], rb(pr2['edge_mlp'][2][1]),
        pr2['edge_fn'][0][0], rb(pr2['edge_fn'][0][1]),
        pr2['edge_fn'][1][0], rb(pr2['edge_fn'][1][1]),
        pr2['edge_fn'][2][0], rb(pr2['edge_fn'][2][1]),
        ed0[0], rb(ed0[1]), ed1[0], rb(ed1[1]))
    agg2 = _sc_scatter(msg2, src_r, zeros_nodes)
    node_out = _node_fn(
        True, h2, agg2[0], agg2[1],
        nf2_w0[:D], nf2_w0[D:], rb(nf2_b0),
        pr2['node_fn'][1][0], rb(pr2['node_fn'][1][1]),
        pr2['node_fn'][2][0], rb(pr2['node_fn'][2][1]),
        nd0[0], rb(nd0[1]), nd1[0], rb(nd1[1]))[0]

    return (node_out[:N_NODES], edge_out[:N_EDGES])


# speedup vs baseline: 1.9474x; 1.1026x over previous
import functools

import jax
import jax.numpy as jnp
from jax import lax
from jax.experimental import pallas as pl
from jax.experimental.pallas import tpu as pltpu
from jax.experimental.pallas import tpu_sc as plsc

N_NODES = 10000
N_EDGES = 320000
D = 128

NN_PAD = 10240
NE_PAD = 327680
NW = 32
NCH = 80
CH = 128
EPW = NCH * CH

BN = 1024
BE = 2048

_F32 = jnp.float32


def _mish(v):
    return v * jnp.tanh(jax.nn.softplus(v))


def _relu(v):
    return jnp.maximum(v, 0.0)


def _dot(a, b):
    return jnp.dot(a, b, preferred_element_type=_F32)


def _wspec(r, c):
    return pl.BlockSpec((r, c), lambda i: (0, 0))


def _rspec(rows, cols):
    return pl.BlockSpec((rows, cols), lambda i: (i, 0))


def _node_enc_body(x_ref, w0, b0, w1, b1, wa, wb, h_ref, a_ref, b_ref):
    t = _mish(_dot(x_ref[...], w0[...]) + b0[...])
    h = _dot(t, w1[...]) + b1[...]
    h_ref[...] = h
    a_ref[...] = _dot(h, wa[...])
    b_ref[...] = _dot(h, wb[...])


def _node_encode(x_p, w0, b0, w1, b1, wa, wb):
    return pl.pallas_call(
        _node_enc_body,
        grid=(NN_PAD // BN,),
        in_specs=[_rspec(BN, D), _wspec(D, D), _wspec(1, D), _wspec(D, D),
                  _wspec(1, D), _wspec(D, D), _wspec(D, D)],
        out_specs=[_rspec(BN, D)] * 3,
        out_shape=[jax.ShapeDtypeStruct((NN_PAD, D), _F32)] * 3,
    )(x_p, w0, b0, w1, b1, wa, wb)


def _edge_enc_body(ea_ref, e0, e0b, e1, e1b, wc, cb, c_ref):
    t = _mish(_dot(ea_ref[...], e0[...]) + e0b[...])
    e = _dot(t, e1[...]) + e1b[...]
    c_ref[...] = _dot(e, wc[...]) + cb[...]


def _edge_encode(ea_p, e0, e0b, e1, e1b, wc, cb):
    return pl.pallas_call(
        _edge_enc_body,
        grid=(NE_PAD // BE,),
        in_specs=[_rspec(BE, 4), _wspec(4, D), _wspec(1, D), _wspec(D, D),
                  _wspec(1, D), _wspec(D, D), _wspec(1, D)],
        out_specs=_rspec(BE, D),
        out_shape=jax.ShapeDtypeStruct((NE_PAD, D), _F32),
    )(ea_p, e0, e0b, e1, e1b, wc, cb)


def _edge_mega_body(final, ga, gb, c, w2, b2, w3, b3,
                    f0, f0b, f1, f1b, f2, f2b, t0, t0b, t1, t1b,
                    msg_ref, tail_ref):
    z1 = _relu(ga[...] + gb[...] + c[...])
    h2 = _relu(_dot(z1, w2[...]) + b2[...])
    msg = _dot(h2, w3[...]) + b3[...]
    msg_ref[...] = msg
    f = _relu(_dot(msg, f0[...]) + f0b[...])
    f = _relu(_dot(f, f1[...]) + f1b[...])
    e2 = _dot(f, f2[...]) + f2b[...]
    if final:
        d = _mish(_dot(e2, t0[...]) + t0b[...])
        tail_ref[...] = _dot(d, t1[...]) + t1b[...]
    else:
        tail_ref[...] = _dot(e2, t0[...]) + t0b[...]


def _edge_mega(final, ga, gb, c, w2, b2, w3, b3,
               f0, f0b, f1, f1b, f2, f2b, t0, t0b, t1, t1b):
    tail_w = 4 if final else D
    return pl.pallas_call(
        functools.partial(_edge_mega_body, final),
        grid=(NE_PAD // BE,),
        in_specs=[_rspec(BE, D)] * 3 + [
            _wspec(D, D), _wspec(1, D), _wspec(D, D), _wspec(1, D),
            _wspec(D, D), _wspec(1, D), _wspec(D, D), _wspec(1, D),
            _wspec(D, D), _wspec(1, D),
            _wspec(D, t0.shape[1]), _wspec(1, t0b.shape[1]),
            _wspec(t1.shape[0], t1.shape[1]), _wspec(1, t1b.shape[1])],
        out_specs=[_rspec(BE, D), _rspec(BE, tail_w)],
        out_shape=[jax.ShapeDtypeStruct((NE_PAD, D), _F32),
                   jax.ShapeDtypeStruct((NE_PAD, tail_w), _F32)],
    )(ga, gb, c, w2, b2, w3, b3, f0, f0b, f1, f1b, f2, f2b, t0, t0b, t1, t1b)


def _node_fn_body(final, h_ref, ag0, ag1, na, nb, n0b, n1, n1b, n2, n2b,
                  t0, t0b, t1, t1b, *out_refs):
    agg = ag0[...] + ag1[...]
    n = _relu(_dot(h_ref[...], na[...]) + _dot(agg, nb[...]) + n0b[...])
    n = _relu(_dot(n, n1[...]) + n1b[...])
    h2 = _dot(n, n2[...]) + n2b[...]
    if final:
        d = _mish(_dot(h2, t0[...]) + t0b[...])
        out_refs[0][...] = _dot(d, t1[...]) + t1b[...]
    else:
        out_refs[0][...] = h2
        out_refs[1][...] = _dot(h2, t0[...])
        out_refs[2][...] = _dot(h2, t1[...])


def _node_fn(final, h, ag0, ag1, na, nb, n0b, n1, n1b, n2, n2b,
             t0, t0b, t1, t1b):
    if final:
        out_specs = [_rspec(BN, 3)]
        out_shape = [jax.ShapeDtypeStruct((NN_PAD, 3), _F32)]
    else:
        out_specs = [_rspec(BN, D)] * 3
        out_shape = [jax.ShapeDtypeStruct((NN_PAD, D), _F32)] * 3
    res = pl.pallas_call(
        functools.partial(_node_fn_body, final),
        grid=(NN_PAD // BN,),
        in_specs=[_rspec(BN, D)] * 3 + [
            _wspec(D, D), _wspec(D, D), _wspec(1, D), _wspec(D, D),
            _wspec(1, D), _wspec(D, D), _wspec(1, D),
            _wspec(D, t0.shape[1]), _wspec(1, t0b.shape[1]),
            _wspec(t1.shape[0], t1.shape[1]), _wspec(1, t1b.shape[1])],
        out_specs=out_specs,
        out_shape=out_shape,
    )(h, ag0, ag1, na, nb, n0b, n1, n1b, n2, n2b, t0, t0b, t1, t1b)
    return res


def _sc_mesh():
    return plsc.VectorSubcoreMesh(core_axis_name="c", subcore_axis_name="s")


NB = 2


def _sc_gather(a_pad, b_pad, dst_r, src_r):
    @functools.partial(
        pl.kernel,
        out_type=[jax.ShapeDtypeStruct((NE_PAD, D), _F32),
                  jax.ShapeDtypeStruct((NE_PAD, D), _F32)],
        mesh=_sc_mesh(),
        scratch_types=[
            pltpu.VMEM((NCH, CH), jnp.int32),
            pltpu.VMEM((NCH, CH), jnp.int32),
            pltpu.VMEM((NB, CH, D), _F32),
            pltpu.VMEM((NB, CH, D), _F32),
        ] + [pltpu.SemaphoreType.DMA] * (4 * NB),
    )
    def k(a_hbm, b_hbm, dst_hbm, src_hbm, ga_hbm, gb_hbm,
          dst_v, src_v, bufa, bufb, *sems):
        ga_sem = sems[0:NB]
        gb_sem = sems[NB:2 * NB]
        wa_sem = sems[2 * NB:3 * NB]
        wb_sem = sems[3 * NB:4 * NB]
        wid = lax.axis_index("s") * 2 + lax.axis_index("c")
        base = wid * EPW
        pltpu.sync_copy(dst_hbm.at[wid], dst_v)
        pltpu.sync_copy(src_hbm.at[wid], src_v)
        for b in range(NB):
            pltpu.async_copy(a_hbm.at[dst_v.at[b]], bufa.at[b], ga_sem[b])
            pltpu.async_copy(b_hbm.at[src_v.at[b]], bufb.at[b], gb_sem[b])

        def g_body(g, carry):
            for b in range(NB):
                j = g * NB + b
                off = base + j * CH
                pltpu.make_async_copy(a_hbm.at[dst_v.at[j]], bufa.at[b],
                                      ga_sem[b]).wait()
                pltpu.make_async_copy(b_hbm.at[src_v.at[j]], bufb.at[b],
                                      gb_sem[b]).wait()
                pltpu.async_copy(bufa.at[b], ga_hbm.at[pl.ds(off, CH)],
                                 wa_sem[b])
                pltpu.async_copy(bufb.at[b], gb_hbm.at[pl.ds(off, CH)],
                                 wb_sem[b])
                nxt = j + NB

                @pl.when(nxt < NCH)
                def _():
                    pltpu.make_async_copy(bufa.at[b],
                                          ga_hbm.at[pl.ds(off, CH)],
                                          wa_sem[b]).wait()
                    pltpu.make_async_copy(bufb.at[b],
                                          gb_hbm.at[pl.ds(off, CH)],
                                          wb_sem[b]).wait()
                    pltpu.async_copy(a_hbm.at[dst_v.at[nxt]], bufa.at[b],
                                     ga_sem[b])
                    pltpu.async_copy(b_hbm.at[src_v.at[nxt]], bufb.at[b],
                                     gb_sem[b])
            return carry

        lax.fori_loop(0, NCH // NB, g_body, 0)
        for b in range(NB):
            pltpu.make_async_copy(bufa.at[b], ga_hbm.at[pl.ds(base, CH)],
                                  wa_sem[b]).wait()
            pltpu.make_async_copy(bufb.at[b], gb_hbm.at[pl.ds(base, CH)],
                                  wb_sem[b]).wait()

    return k(a_pad, b_pad, dst_r, src_r)


def _sc_scatter(msg, idx_r, zeros_nodes):
    rows_per_tile = NN_PAD // 16

    @functools.partial(
        pl.kernel,
        out_type=jax.ShapeDtypeStruct((2, NN_PAD, D), _F32),
        mesh=_sc_mesh(),
        scratch_types=[
            pltpu.VMEM((NCH, CH), jnp.int32),
            pltpu.VMEM((NB, CH, D), _F32),
            pltpu.VMEM_SHARED((NN_PAD, D), _F32),
        ] + [pltpu.SemaphoreType.DMA] * NB,
    )
    def k(msg_hbm, idx_hbm, zer_hbm, out_hbm, idx_v, rows_v, acc, *rsem):
        cid = lax.axis_index("c")
        sid = lax.axis_index("s")
        wid = sid * 2 + cid
        base = wid * EPW
        pltpu.sync_copy(idx_hbm.at[wid], idx_v)
        pltpu.sync_copy(zer_hbm.at[pl.ds(sid * rows_per_tile, rows_per_tile)],
                        acc.at[pl.ds(sid * rows_per_tile, rows_per_tile)])
        plsc.subcore_barrier()
        for b in range(NB):
            pltpu.async_copy(msg_hbm.at[pl.ds(base + b * CH, CH)],
                             rows_v.at[b], rsem[b])

        def chunk(g, carry):
            for b in range(NB):
                j = g * NB + b
                pltpu.make_async_copy(msg_hbm.at[pl.ds(base + j * CH, CH)],
                                      rows_v.at[b], rsem[b]).wait()
                pltpu.sync_copy(rows_v.at[b], acc.at[idx_v.at[j]], add=True)
                nxt = j + NB

                @pl.when(nxt < NCH)
                def _():
                    pltpu.async_copy(msg_hbm.at[pl.ds(base + nxt * CH, CH)],
                                     rows_v.at[b], rsem[b])
            return carry

        lax.fori_loop(0, NCH // NB, chunk, 0)
        plsc.subcore_barrier()
        pltpu.sync_copy(acc.at[pl.ds(sid * rows_per_tile, rows_per_tile)],
                        out_hbm.at[cid, pl.ds(sid * rows_per_tile, rows_per_tile)])

    return k(msg, idx_r, zeros_nodes)


def kernel(x, edge_index, edge_attr, params):
    ne0, ne1 = params['node_encoder']
    ee0, ee1 = params['edge_encoder']
    pr1, pr2 = params['processors']
    nd0, nd1 = params['node_decoder']
    ed0, ed1 = params['edge_decoder']

    def rb(b):
        return b.reshape(1, -1)

    em1_w0, em1_b0 = pr1['edge_mlp'][0]
    em2_w0, em2_b0 = pr2['edge_mlp'][0]
    w1a_1, w1b_1, w1c_1 = em1_w0[:D], em1_w0[D:2 * D], em1_w0[2 * D:]
    w1a_2, w1b_2, w1c_2 = em2_w0[:D], em2_w0[D:2 * D], em2_w0[2 * D:]

    nf1_w0, nf1_b0 = pr1['node_fn'][0]
    nf2_w0, nf2_b0 = pr2['node_fn'][0]

    src = edge_index[0]
    dst = edge_index[1]
    pad_e = NE_PAD - N_EDGES
    src_r = jnp.pad(src, (0, pad_e), constant_values=N_NODES).reshape(NW, NCH, CH)
    dst_r = jnp.pad(dst, (0, pad_e), constant_values=N_NODES).reshape(NW, NCH, CH)
    x_p = jnp.pad(x, ((0, NN_PAD - N_NODES), (0, 0)))
    ea_p = jnp.pad(edge_attr, ((0, pad_e), (0, 0)))
    zeros_nodes = jnp.zeros((NN_PAD, D), _F32)

    h, a1, b1 = _node_encode(x_p, ne0[0], rb(ne0[1]), ne1[0], rb(ne1[1]),
                             w1a_1, w1b_1)
    c1 = _edge_encode(ea_p, ee0[0], rb(ee0[1]), ee1[0], rb(ee1[1]),
                      w1c_1, rb(em1_b0))

    ga1, gb1 = _sc_gather(a1, b1, dst_r, src_r)
    msg1, c2 = _edge_mega(
        False, ga1, gb1, c1,
        pr1['edge_mlp'][1][0], rb(pr1['edge_mlp'][1][1]),
        pr1['edge_mlp'][2][0], rb(pr1['edge_mlp'][2][1]),
        pr1['edge_fn'][0][0], rb(pr1['edge_fn'][0][1]),
        pr1['edge_fn'][1][0], rb(pr1['edge_fn'][1][1]),
        pr1['edge_fn'][2][0], rb(pr1['edge_fn'][2][1]),
        w1c_2, rb(em2_b0), w1c_2, rb(em2_b0))
    agg1 = _sc_scatter(msg1, src_r, zeros_nodes)
    h2, a2, b2 = _node_fn(
        False, h, agg1[0], agg1[1],
        nf1_w0[:D], nf1_w0[D:], rb(nf1_b0),
        pr1['node_fn'][1][0], rb(pr1['node_fn'][1][1]),
        pr1['node_fn'][2][0], rb(pr1['node_fn'][2][1]),
        w1a_2, rb(em2_b0), w1b_2, rb(em2_b0))

    ga2, gb2 = _sc_gather(a2, b2, dst_r, src_r)
    msg2, edge_out = _edge_mega(
        True, ga2, gb2, c2,
        pr2['edge_mlp'][1][0], rb(pr2['edge_mlp'][1][1]),
        pr2['edge_mlp'][2][0], rb(pr2['edge_mlp'][2][1]),
        pr2['edge_fn'][0][0], rb(pr2['edge_fn'][0][1]),
        pr2['edge_fn'][1][0], rb(pr2['edge_fn'][1][1]),
        pr2['edge_fn'][2][0], rb(pr2['edge_fn'][2][1]),
        ed0[0], rb(ed0[1]), ed1[0], rb(ed1[1]))
    agg2 = _sc_scatter(msg2, src_r, zeros_nodes)
    node_out = _node_fn(
        True, h2, agg2[0], agg2[1],
        nf2_w0[:D], nf2_w0[D:], rb(nf2_b0),
        pr2['node_fn'][1][0], rb(pr2['node_fn'][1][1]),
        pr2['node_fn'][2][0], rb(pr2['node_fn'][2][1]),
        nd0[0], rb(nd0[1]), nd1[0], rb(nd1[1]))[0]

    return (node_out[:N_NODES], edge_out[:N_EDGES])


# speedup vs baseline: 2.7886x; 1.4320x over previous
import functools

import jax
import jax.numpy as jnp
from jax import lax
from jax.experimental import pallas as pl
from jax.experimental.pallas import tpu as pltpu
from jax.experimental.pallas import tpu_sc as plsc

N_NODES = 10000
N_EDGES = 320000
D = 128

NN_PAD = 10240
NE_PAD = 327680
NW = 32
NCH = 80
CH = 128
EPW = NCH * CH

BN = 1024
BE = 2048

_F32 = jnp.float32


def _mish(v):
    return v * jnp.tanh(jax.nn.softplus(v))


def _relu(v):
    return jnp.maximum(v, 0.0)


def _dot(a, b):
    return jnp.dot(a, b, preferred_element_type=_F32)


def _wspec(r, c):
    return pl.BlockSpec((r, c), lambda i: (0, 0))


def _rspec(rows, cols):
    return pl.BlockSpec((rows, cols), lambda i: (i, 0))


def _node_enc_body(x_ref, w0, b0, w1, b1, wa, wb, h_ref, a_ref, b_ref):
    t = _mish(_dot(x_ref[...], w0[...]) + b0[...])
    h = _dot(t, w1[...]) + b1[...]
    h_ref[...] = h
    a_ref[...] = _dot(h, wa[...])
    b_ref[...] = _dot(h, wb[...])


def _node_encode(x_p, w0, b0, w1, b1, wa, wb):
    return pl.pallas_call(
        _node_enc_body,
        grid=(NN_PAD // BN,),
        in_specs=[_rspec(BN, D), _wspec(D, D), _wspec(1, D), _wspec(D, D),
                  _wspec(1, D), _wspec(D, D), _wspec(D, D)],
        out_specs=[_rspec(BN, D)] * 3,
        out_shape=[jax.ShapeDtypeStruct((NN_PAD, D), _F32)] * 3,
    )(x_p, w0, b0, w1, b1, wa, wb)


def _edge_enc_body(ea_ref, e0, e0b, e1, e1b, wc, cb, c_ref):
    t = _mish(_dot(ea_ref[...], e0[...]) + e0b[...])
    e = _dot(t, e1[...]) + e1b[...]
    c_ref[...] = _dot(e, wc[...]) + cb[...]


def _edge_encode(ea_p, e0, e0b, e1, e1b, wc, cb):
    return pl.pallas_call(
        _edge_enc_body,
        grid=(NE_PAD // BE,),
        in_specs=[_rspec(BE, 4), _wspec(4, D), _wspec(1, D), _wspec(D, D),
                  _wspec(1, D), _wspec(D, D), _wspec(1, D)],
        out_specs=_rspec(BE, D),
        out_shape=jax.ShapeDtypeStruct((NE_PAD, D), _F32),
    )(ea_p, e0, e0b, e1, e1b, wc, cb)


def _edge_mega_body(final, s, c, w2, b2, w3, b3,
                    f0, f0b, f1, f1b, f2, f2b, t0, t0b, t1, t1b,
                    msg_ref, tail_ref):
    z1 = _relu(s[...] + c[...])
    h2 = _relu(_dot(z1, w2[...]) + b2[...])
    msg = _dot(h2, w3[...]) + b3[...]
    msg_ref[...] = msg
    f = _relu(_dot(msg, f0[...]) + f0b[...])
    f = _relu(_dot(f, f1[...]) + f1b[...])
    e2 = _dot(f, f2[...]) + f2b[...]
    if final:
        d = _mish(_dot(e2, t0[...]) + t0b[...])
        tail_ref[...] = _dot(d, t1[...]) + t1b[...]
    else:
        tail_ref[...] = _dot(e2, t0[...]) + t0b[...]


def _edge_mega(final, s, c, w2, b2, w3, b3,
               f0, f0b, f1, f1b, f2, f2b, t0, t0b, t1, t1b):
    tail_w = 4 if final else D
    return pl.pallas_call(
        functools.partial(_edge_mega_body, final),
        grid=(NE_PAD // BE,),
        in_specs=[_rspec(BE, D)] * 2 + [
            _wspec(D, D), _wspec(1, D), _wspec(D, D), _wspec(1, D),
            _wspec(D, D), _wspec(1, D), _wspec(D, D), _wspec(1, D),
            _wspec(D, D), _wspec(1, D),
            _wspec(D, t0.shape[1]), _wspec(1, t0b.shape[1]),
            _wspec(t1.shape[0], t1.shape[1]), _wspec(1, t1b.shape[1])],
        out_specs=[_rspec(BE, D), _rspec(BE, tail_w)],
        out_shape=[jax.ShapeDtypeStruct((NE_PAD, D), _F32),
                   jax.ShapeDtypeStruct((NE_PAD, tail_w), _F32)],
    )(s, c, w2, b2, w3, b3, f0, f0b, f1, f1b, f2, f2b, t0, t0b, t1, t1b)


def _node_fn_body(final, h_ref, ag0, ag1, na, nb, n0b, n1, n1b, n2, n2b,
                  t0, t0b, t1, t1b, *out_refs):
    agg = ag0[...] + ag1[...]
    n = _relu(_dot(h_ref[...], na[...]) + _dot(agg, nb[...]) + n0b[...])
    n = _relu(_dot(n, n1[...]) + n1b[...])
    h2 = _dot(n, n2[...]) + n2b[...]
    if final:
        d = _mish(_dot(h2, t0[...]) + t0b[...])
        out_refs[0][...] = _dot(d, t1[...]) + t1b[...]
    else:
        out_refs[0][...] = h2
        out_refs[1][...] = _dot(h2, t0[...])
        out_refs[2][...] = _dot(h2, t1[...])


def _node_fn(final, h, ag0, ag1, na, nb, n0b, n1, n1b, n2, n2b,
             t0, t0b, t1, t1b):
    if final:
        out_specs = [_rspec(BN, 3)]
        out_shape = [jax.ShapeDtypeStruct((NN_PAD, 3), _F32)]
    else:
        out_specs = [_rspec(BN, D)] * 3
        out_shape = [jax.ShapeDtypeStruct((NN_PAD, D), _F32)] * 3
    res = pl.pallas_call(
        functools.partial(_node_fn_body, final),
        grid=(NN_PAD // BN,),
        in_specs=[_rspec(BN, D)] * 3 + [
            _wspec(D, D), _wspec(D, D), _wspec(1, D), _wspec(D, D),
            _wspec(1, D), _wspec(D, D), _wspec(1, D),
            _wspec(D, t0.shape[1]), _wspec(1, t0b.shape[1]),
            _wspec(t1.shape[0], t1.shape[1]), _wspec(1, t1b.shape[1])],
        out_specs=out_specs,
        out_shape=out_shape,
    )(h, ag0, ag1, na, nb, n0b, n1, n1b, n2, n2b, t0, t0b, t1, t1b)
    return res


def _sc_mesh():
    return plsc.VectorSubcoreMesh(core_axis_name="c", subcore_axis_name="s")


NB = 2


def _sc_gather(a_pad, b_pad, dst_r, src_r):
    @functools.partial(
        pl.kernel,
        out_type=jax.ShapeDtypeStruct((NE_PAD, D), _F32),
        mesh=_sc_mesh(),
        scratch_types=[
            pltpu.VMEM((NCH, CH), jnp.int32),
            pltpu.VMEM((NCH, CH), jnp.int32),
            pltpu.VMEM((NB, CH, D), _F32),
        ] + [pltpu.SemaphoreType.DMA] * (2 * NB),
    )
    def k(a_hbm, b_hbm, dst_hbm, src_hbm, s_hbm,
          dst_v, src_v, buf, *sems):
        g_sem = sems[0:NB]
        w_sem = sems[NB:2 * NB]
        wid = lax.axis_index("s") * 2 + lax.axis_index("c")
        base = wid * EPW
        pltpu.sync_copy(dst_hbm.at[wid], dst_v)
        pltpu.sync_copy(src_hbm.at[wid], src_v)
        for b in range(NB):
            pltpu.async_copy(a_hbm.at[dst_v.at[b]], buf.at[b], g_sem[b])

        def g_body(g, carry):
            for b in range(NB):
                j = g * NB + b
                off = base + j * CH
                pltpu.make_async_copy(a_hbm.at[dst_v.at[j]], buf.at[b],
                                      g_sem[b]).wait()
                pltpu.sync_copy(b_hbm.at[src_v.at[j]], buf.at[b], add=True)
                pltpu.async_copy(buf.at[b], s_hbm.at[pl.ds(off, CH)],
                                 w_sem[b])
                nxt = j + NB

                @pl.when(nxt < NCH)
                def _():
                    pltpu.make_async_copy(buf.at[b],
                                          s_hbm.at[pl.ds(off, CH)],
                                          w_sem[b]).wait()
                    pltpu.async_copy(a_hbm.at[dst_v.at[nxt]], buf.at[b],
                                     g_sem[b])
            return carry

        lax.fori_loop(0, NCH // NB, g_body, 0)
        for b in range(NB):
            pltpu.make_async_copy(buf.at[b], s_hbm.at[pl.ds(base, CH)],
                                  w_sem[b]).wait()

    return k(a_pad, b_pad, dst_r, src_r)


def _sc_scatter(msg, idx_r, zeros_nodes):
    rows_per_tile = NN_PAD // 16

    @functools.partial(
        pl.kernel,
        out_type=jax.ShapeDtypeStruct((2, NN_PAD, D), _F32),
        mesh=_sc_mesh(),
        scratch_types=[
            pltpu.VMEM((NCH, CH), jnp.int32),
            pltpu.VMEM((NB, CH, D), _F32),
            pltpu.VMEM_SHARED((NN_PAD, D), _F32),
        ] + [pltpu.SemaphoreType.DMA] * NB,
    )
    def k(msg_hbm, idx_hbm, zer_hbm, out_hbm, idx_v, rows_v, acc, *rsem):
        cid = lax.axis_index("c")
        sid = lax.axis_index("s")
        wid = sid * 2 + cid
        base = wid * EPW
        pltpu.sync_copy(idx_hbm.at[wid], idx_v)
        pltpu.sync_copy(zer_hbm.at[pl.ds(sid * rows_per_tile, rows_per_tile)],
                        acc.at[pl.ds(sid * rows_per_tile, rows_per_tile)])
        plsc.subcore_barrier()
        for b in range(NB):
            pltpu.async_copy(msg_hbm.at[pl.ds(base + b * CH, CH)],
                             rows_v.at[b], rsem[b])

        def chunk(g, carry):
            for b in range(NB):
                j = g * NB + b
                pltpu.make_async_copy(msg_hbm.at[pl.ds(base + j * CH, CH)],
                                      rows_v.at[b], rsem[b]).wait()
                pltpu.sync_copy(rows_v.at[b], acc.at[idx_v.at[j]], add=True)
                nxt = j + NB

                @pl.when(nxt < NCH)
                def _():
                    pltpu.async_copy(msg_hbm.at[pl.ds(base + nxt * CH, CH)],
                                     rows_v.at[b], rsem[b])
            return carry

        lax.fori_loop(0, NCH // NB, chunk, 0)
        plsc.subcore_barrier()
        pltpu.sync_copy(acc.at[pl.ds(sid * rows_per_tile, rows_per_tile)],
                        out_hbm.at[cid, pl.ds(sid * rows_per_tile, rows_per_tile)])

    return k(msg, idx_r, zeros_nodes)


def kernel(x, edge_index, edge_attr, params):
    ne0, ne1 = params['node_encoder']
    ee0, ee1 = params['edge_encoder']
    pr1, pr2 = params['processors']
    nd0, nd1 = params['node_decoder']
    ed0, ed1 = params['edge_decoder']

    def rb(b):
        return b.reshape(1, -1)

    em1_w0, em1_b0 = pr1['edge_mlp'][0]
    em2_w0, em2_b0 = pr2['edge_mlp'][0]
    w1a_1, w1b_1, w1c_1 = em1_w0[:D], em1_w0[D:2 * D], em1_w0[2 * D:]
    w1a_2, w1b_2, w1c_2 = em2_w0[:D], em2_w0[D:2 * D], em2_w0[2 * D:]

    nf1_w0, nf1_b0 = pr1['node_fn'][0]
    nf2_w0, nf2_b0 = pr2['node_fn'][0]

    src = edge_index[0]
    dst = edge_index[1]
    pad_e = NE_PAD - N_EDGES
    pad_idx = N_NODES + (jnp.arange(pad_e, dtype=jnp.int32)
                         % (NN_PAD - N_NODES))
    src_r = jnp.concatenate([src, pad_idx]).reshape(NW, NCH, CH)
    dst_r = jnp.concatenate([dst, pad_idx]).reshape(NW, NCH, CH)
    x_p = jnp.pad(x, ((0, NN_PAD - N_NODES), (0, 0)))
    ea_p = jnp.pad(edge_attr, ((0, pad_e), (0, 0)))
    zeros_nodes = jnp.zeros((NN_PAD, D), _F32)

    h, a1, b1 = _node_encode(x_p, ne0[0], rb(ne0[1]), ne1[0], rb(ne1[1]),
                             w1a_1, w1b_1)
    c1 = _edge_encode(ea_p, ee0[0], rb(ee0[1]), ee1[0], rb(ee1[1]),
                      w1c_1, rb(em1_b0))

    s1 = _sc_gather(a1, b1, dst_r, src_r)
    msg1, c2 = _edge_mega(
        False, s1, c1,
        pr1['edge_mlp'][1][0], rb(pr1['edge_mlp'][1][1]),
        pr1['edge_mlp'][2][0], rb(pr1['edge_mlp'][2][1]),
        pr1['edge_fn'][0][0], rb(pr1['edge_fn'][0][1]),
        pr1['edge_fn'][1][0], rb(pr1['edge_fn'][1][1]),
        pr1['edge_fn'][2][0], rb(pr1['edge_fn'][2][1]),
        w1c_2, rb(em2_b0), w1c_2, rb(em2_b0))
    agg1 = _sc_scatter(msg1, src_r, zeros_nodes)
    h2, a2, b2 = _node_fn(
        False, h, agg1[0], agg1[1],
        nf1_w0[:D], nf1_w0[D:], rb(nf1_b0),
        pr1['node_fn'][1][0], rb(pr1['node_fn'][1][1]),
        pr1['node_fn'][2][0], rb(pr1['node_fn'][2][1]),
        w1a_2, rb(em2_b0), w1b_2, rb(em2_b0))

    s2 = _sc_gather(a2, b2, dst_r, src_r)
    msg2, edge_out = _edge_mega(
        True, s2, c2,
        pr2['edge_mlp'][1][0], rb(pr2['edge_mlp'][1][1]),
        pr2['edge_mlp'][2][0], rb(pr2['edge_mlp'][2][1]),
        pr2['edge_fn'][0][0], rb(pr2['edge_fn'][0][1]),
        pr2['edge_fn'][1][0], rb(pr2['edge_fn'][1][1]),
        pr2['edge_fn'][2][0], rb(pr2['edge_fn'][2][1]),
        ed0[0], rb(ed0[1]), ed1[0], rb(ed1[1]))
    agg2 = _sc_scatter(msg2, src_r, zeros_nodes)
    node_out = _node_fn(
        True, h2, agg2[0], agg2[1],
        nf2_w0[:D], nf2_w0[D:], rb(nf2_b0),
        pr2['node_fn'][1][0], rb(pr2['node_fn'][1][1]),
        pr2['node_fn'][2][0], rb(pr2['node_fn'][2][1]),
        nd0[0], rb(nd0[1]), nd1[0], rb(nd1[1]))[0]

    return (node_out[:N_NODES], edge_out[:N_EDGES])


# speedup vs baseline: 2.8739x; 1.0306x over previous
import functools

import jax
import jax.numpy as jnp
from jax import lax
from jax.experimental import pallas as pl
from jax.experimental.pallas import tpu as pltpu
from jax.experimental.pallas import tpu_sc as plsc

N_NODES = 10000
N_EDGES = 320000
D = 128

NN_PAD = 10240
NE_PAD = 327680
NW = 32
NCH = 80
CH = 128
EPW = NCH * CH

BN = 1024
BE = 2048

_F32 = jnp.float32
_BF16 = jnp.bfloat16


def _mish(v):
    return v * jnp.tanh(jax.nn.softplus(v))


def _relu(v):
    return jnp.maximum(v, 0.0)


def _dot(a, b):
    return jnp.dot(a, b, preferred_element_type=_F32)


def _wspec(r, c):
    return pl.BlockSpec((r, c), lambda i: (0, 0))


def _rspec(rows, cols):
    return pl.BlockSpec((rows, cols), lambda i: (i, 0))


def _node_enc_body(x_ref, w0, b0, w1, b1, wa, wb, h_ref, a_ref, b_ref):
    t = _mish(_dot(x_ref[...], w0[...]) + b0[...])
    h = _dot(t, w1[...]) + b1[...]
    h_ref[...] = h
    a_ref[...] = _dot(h, wa[...])
    b_ref[...] = _dot(h, wb[...])


def _node_encode(x_p, w0, b0, w1, b1, wa, wb):
    return pl.pallas_call(
        _node_enc_body,
        grid=(NN_PAD // BN,),
        in_specs=[_rspec(BN, D), _wspec(D, D), _wspec(1, D), _wspec(D, D),
                  _wspec(1, D), _wspec(D, D), _wspec(D, D)],
        out_specs=[_rspec(BN, D)] * 3,
        out_shape=[jax.ShapeDtypeStruct((NN_PAD, D), _F32)] * 3,
    )(x_p, w0, b0, w1, b1, wa, wb)


def _edge_enc_body(ea_ref, e0, e0b, e1, e1b, wc, cb, c_ref):
    t = _mish(_dot(ea_ref[...], e0[...]) + e0b[...])
    e = _dot(t, e1[...]) + e1b[...]
    c_ref[...] = (_dot(e, wc[...]) + cb[...]).astype(_BF16)


def _edge_encode(ea_p, e0, e0b, e1, e1b, wc, cb):
    return pl.pallas_call(
        _edge_enc_body,
        grid=(NE_PAD // BE,),
        in_specs=[_rspec(BE, 4), _wspec(4, D), _wspec(1, D), _wspec(D, D),
                  _wspec(1, D), _wspec(D, D), _wspec(1, D)],
        out_specs=_rspec(BE, D),
        out_shape=jax.ShapeDtypeStruct((NE_PAD, D), _BF16),
    )(ea_p, e0, e0b, e1, e1b, wc, cb)


def _edge_mega_body(final, s, c, w2, b2, w3, b3,
                    f0, f0b, f1, f1b, f2, f2b, t0, t0b, t1, t1b,
                    msg_ref, tail_ref):
    z1 = _relu(s[...].astype(_F32) + c[...].astype(_F32))
    h2 = _relu(_dot(z1, w2[...]) + b2[...])
    msg = _dot(h2, w3[...]) + b3[...]
    msg_ref[...] = msg
    f = _relu(_dot(msg, f0[...]) + f0b[...])
    f = _relu(_dot(f, f1[...]) + f1b[...])
    e2 = _dot(f, f2[...]) + f2b[...]
    if final:
        d = _mish(_dot(e2, t0[...]) + t0b[...])
        tail_ref[...] = _dot(d, t1[...]) + t1b[...]
    else:
        tail_ref[...] = (_dot(e2, t0[...]) + t0b[...]).astype(_BF16)


def _edge_mega(final, s, c, w2, b2, w3, b3,
               f0, f0b, f1, f1b, f2, f2b, t0, t0b, t1, t1b):
    tail_w = 4 if final else D
    return pl.pallas_call(
        functools.partial(_edge_mega_body, final),
        grid=(NE_PAD // BE,),
        in_specs=[_rspec(BE, D)] * 2 + [
            _wspec(D, D), _wspec(1, D), _wspec(D, D), _wspec(1, D),
            _wspec(D, D), _wspec(1, D), _wspec(D, D), _wspec(1, D),
            _wspec(D, D), _wspec(1, D),
            _wspec(D, t0.shape[1]), _wspec(1, t0b.shape[1]),
            _wspec(t1.shape[0], t1.shape[1]), _wspec(1, t1b.shape[1])],
        out_specs=[_rspec(BE, D), _rspec(BE, tail_w)],
        out_shape=[jax.ShapeDtypeStruct((NE_PAD, D), _F32),
                   jax.ShapeDtypeStruct((NE_PAD, tail_w),
                                        _F32 if final else _BF16)],
    )(s, c, w2, b2, w3, b3, f0, f0b, f1, f1b, f2, f2b, t0, t0b, t1, t1b)


def _node_fn_body(final, h_ref, ag0, ag1, na, nb, n0b, n1, n1b, n2, n2b,
                  t0, t0b, t1, t1b, *out_refs):
    agg = ag0[...] + ag1[...]
    n = _relu(_dot(h_ref[...], na[...]) + _dot(agg, nb[...]) + n0b[...])
    n = _relu(_dot(n, n1[...]) + n1b[...])
    h2 = _dot(n, n2[...]) + n2b[...]
    if final:
        d = _mish(_dot(h2, t0[...]) + t0b[...])
        out_refs[0][...] = _dot(d, t1[...]) + t1b[...]
    else:
        out_refs[0][...] = h2
        out_refs[1][...] = _dot(h2, t0[...])
        out_refs[2][...] = _dot(h2, t1[...])


def _node_fn(final, h, ag0, ag1, na, nb, n0b, n1, n1b, n2, n2b,
             t0, t0b, t1, t1b):
    if final:
        out_specs = [_rspec(BN, 3)]
        out_shape = [jax.ShapeDtypeStruct((NN_PAD, 3), _F32)]
    else:
        out_specs = [_rspec(BN, D)] * 3
        out_shape = [jax.ShapeDtypeStruct((NN_PAD, D), _F32)] * 3
    res = pl.pallas_call(
        functools.partial(_node_fn_body, final),
        grid=(NN_PAD // BN,),
        in_specs=[_rspec(BN, D)] * 3 + [
            _wspec(D, D), _wspec(D, D), _wspec(1, D), _wspec(D, D),
            _wspec(1, D), _wspec(D, D), _wspec(1, D),
            _wspec(D, t0.shape[1]), _wspec(1, t0b.shape[1]),
            _wspec(t1.shape[0], t1.shape[1]), _wspec(1, t1b.shape[1])],
        out_specs=out_specs,
        out_shape=out_shape,
    )(h, ag0, ag1, na, nb, n0b, n1, n1b, n2, n2b, t0, t0b, t1, t1b)
    return res


def _sc_mesh():
    return plsc.VectorSubcoreMesh(core_axis_name="c", subcore_axis_name="s")


NB = 2


def _sc_gather(a_pad, b_pad, dst_r, src_r):
    @functools.partial(
        pl.kernel,
        out_type=jax.ShapeDtypeStruct((NE_PAD, D), _F32),
        mesh=_sc_mesh(),
        scratch_types=[
            pltpu.VMEM((NCH, CH), jnp.int32),
            pltpu.VMEM((NCH, CH), jnp.int32),
            pltpu.VMEM((NB, CH, D), _F32),
        ] + [pltpu.SemaphoreType.DMA] * (2 * NB),
    )
    def k(a_hbm, b_hbm, dst_hbm, src_hbm, s_hbm,
          dst_v, src_v, buf, *sems):
        g_sem = sems[0:NB]
        w_sem = sems[NB:2 * NB]
        wid = lax.axis_index("s") * 2 + lax.axis_index("c")
        base = wid * EPW
        pltpu.sync_copy(dst_hbm.at[wid], dst_v)
        pltpu.sync_copy(src_hbm.at[wid], src_v)
        for b in range(NB):
            pltpu.async_copy(a_hbm.at[dst_v.at[b]], buf.at[b], g_sem[b])

        def g_body(g, carry):
            for b in range(NB):
                j = g * NB + b
                off = base + j * CH
                pltpu.make_async_copy(a_hbm.at[dst_v.at[j]], buf.at[b],
                                      g_sem[b]).wait()
                pltpu.sync_copy(b_hbm.at[src_v.at[j]], buf.at[b], add=True)
                pltpu.async_copy(buf.at[b], s_hbm.at[pl.ds(off, CH)],
                                 w_sem[b])
                nxt = j + NB

                @pl.when(nxt < NCH)
                def _():
                    pltpu.make_async_copy(buf.at[b],
                                          s_hbm.at[pl.ds(off, CH)],
                                          w_sem[b]).wait()
                    pltpu.async_copy(a_hbm.at[dst_v.at[nxt]], buf.at[b],
                                     g_sem[b])
            return carry

        lax.fori_loop(0, NCH // NB, g_body, 0)
        for b in range(NB):
            pltpu.make_async_copy(buf.at[b], s_hbm.at[pl.ds(base, CH)],
                                  w_sem[b]).wait()

    return k(a_pad, b_pad, dst_r, src_r)


def _sc_scatter(msg, idx_r, zeros_nodes):
    rows_per_tile = NN_PAD // 16

    @functools.partial(
        pl.kernel,
        out_type=jax.ShapeDtypeStruct((2, NN_PAD, D), _F32),
        mesh=_sc_mesh(),
        scratch_types=[
            pltpu.VMEM((NCH, CH), jnp.int32),
            pltpu.VMEM((NB, CH, D), _F32),
            pltpu.VMEM_SHARED((NN_PAD, D), _F32),
        ] + [pltpu.SemaphoreType.DMA] * NB,
    )
    def k(msg_hbm, idx_hbm, zer_hbm, out_hbm, idx_v, rows_v, acc, *rsem):
        cid = lax.axis_index("c")
        sid = lax.axis_index("s")
        wid = sid * 2 + cid
        base = wid * EPW
        pltpu.sync_copy(idx_hbm.at[wid], idx_v)
        pltpu.sync_copy(zer_hbm.at[pl.ds(sid * rows_per_tile, rows_per_tile)],
                        acc.at[pl.ds(sid * rows_per_tile, rows_per_tile)])
        plsc.subcore_barrier()
        for b in range(NB):
            pltpu.async_copy(msg_hbm.at[pl.ds(base + b * CH, CH)],
                             rows_v.at[b], rsem[b])

        def chunk(g, carry):
            for b in range(NB):
                j = g * NB + b
                pltpu.make_async_copy(msg_hbm.at[pl.ds(base + j * CH, CH)],
                                      rows_v.at[b], rsem[b]).wait()
                pltpu.sync_copy(rows_v.at[b], acc.at[idx_v.at[j]], add=True)
                nxt = j + NB

                @pl.when(nxt < NCH)
                def _():
                    pltpu.async_copy(msg_hbm.at[pl.ds(base + nxt * CH, CH)],
                                     rows_v.at[b], rsem[b])
            return carry

        lax.fori_loop(0, NCH // NB, chunk, 0)
        plsc.subcore_barrier()
        pltpu.sync_copy(acc.at[pl.ds(sid * rows_per_tile, rows_per_tile)],
                        out_hbm.at[cid, pl.ds(sid * rows_per_tile, rows_per_tile)])

    return k(msg, idx_r, zeros_nodes)


def kernel(x, edge_index, edge_attr, params):
    ne0, ne1 = params['node_encoder']
    ee0, ee1 = params['edge_encoder']
    pr1, pr2 = params['processors']
    nd0, nd1 = params['node_decoder']
    ed0, ed1 = params['edge_decoder']

    def rb(b):
        return b.reshape(1, -1)

    em1_w0, em1_b0 = pr1['edge_mlp'][0]
    em2_w0, em2_b0 = pr2['edge_mlp'][0]
    w1a_1, w1b_1, w1c_1 = em1_w0[:D], em1_w0[D:2 * D], em1_w0[2 * D:]
    w1a_2, w1b_2, w1c_2 = em2_w0[:D], em2_w0[D:2 * D], em2_w0[2 * D:]

    nf1_w0, nf1_b0 = pr1['node_fn'][0]
    nf2_w0, nf2_b0 = pr2['node_fn'][0]

    src = edge_index[0]
    dst = edge_index[1]
    pad_e = NE_PAD - N_EDGES
    pad_idx = N_NODES + (jnp.arange(pad_e, dtype=jnp.int32)
                         % (NN_PAD - N_NODES))
    src_r = jnp.concatenate([src, pad_idx]).reshape(NW, NCH, CH)
    dst_r = jnp.concatenate([dst, pad_idx]).reshape(NW, NCH, CH)
    x_p = jnp.pad(x, ((0, NN_PAD - N_NODES), (0, 0)))
    ea_p = jnp.pad(edge_attr, ((0, pad_e), (0, 0)))
    zeros_nodes = jnp.zeros((NN_PAD, D), _F32)

    h, a1, b1 = _node_encode(x_p, ne0[0], rb(ne0[1]), ne1[0], rb(ne1[1]),
                             w1a_1, w1b_1)
    c1 = _edge_encode(ea_p, ee0[0], rb(ee0[1]), ee1[0], rb(ee1[1]),
                      w1c_1, rb(em1_b0))

    s1 = _sc_gather(a1, b1, dst_r, src_r)
    msg1, c2 = _edge_mega(
        False, s1, c1,
        pr1['edge_mlp'][1][0], rb(pr1['edge_mlp'][1][1]),
        pr1['edge_mlp'][2][0], rb(pr1['edge_mlp'][2][1]),
        pr1['edge_fn'][0][0], rb(pr1['edge_fn'][0][1]),
        pr1['edge_fn'][1][0], rb(pr1['edge_fn'][1][1]),
        pr1['edge_fn'][2][0], rb(pr1['edge_fn'][2][1]),
        w1c_2, rb(em2_b0), w1c_2, rb(em2_b0))
    agg1 = _sc_scatter(msg1, src_r, zeros_nodes)
    h2, a2, b2 = _node_fn(
        False, h, agg1[0], agg1[1],
        nf1_w0[:D], nf1_w0[D:], rb(nf1_b0),
        pr1['node_fn'][1][0], rb(pr1['node_fn'][1][1]),
        pr1['node_fn'][2][0], rb(pr1['node_fn'][2][1]),
        w1a_2, rb(em2_b0), w1b_2, rb(em2_b0))

    s2 = _sc_gather(a2, b2, dst_r, src_r)
    msg2, edge_out = _edge_mega(
        True, s2, c2,
        pr2['edge_mlp'][1][0], rb(pr2['edge_mlp'][1][1]),
        pr2['edge_mlp'][2][0], rb(pr2['edge_mlp'][2][1]),
        pr2['edge_fn'][0][0], rb(pr2['edge_fn'][0][1]),
        pr2['edge_fn'][1][0], rb(pr2['edge_fn'][1][1]),
        pr2['edge_fn'][2][0], rb(pr2['edge_fn'][2][1]),
        ed0[0], rb(ed0[1]), ed1[0], rb(ed1[1]))
    agg2 = _sc_scatter(msg2, src_r, zeros_nodes)
    node_out = _node_fn(
        True, h2, agg2[0], agg2[1],
        nf2_w0[:D], nf2_w0[D:], rb(nf2_b0),
        pr2['node_fn'][1][0], rb(pr2['node_fn'][1][1]),
        pr2['node_fn'][2][0], rb(pr2['node_fn'][2][1]),
        nd0[0], rb(nd0[1]), nd1[0], rb(nd1[1]))[0]

    return (node_out[:N_NODES], edge_out[:N_EDGES])
